# baseline (speedup 1.0000x reference)
import functools

import jax
import jax.numpy as jnp
from jax import lax
from jax.experimental import pallas as pl
from jax.experimental.pallas import tpu as pltpu
from jax.experimental.pallas import tpu_sc as plsc

NC = 2
NS = 16
NW = NC * NS
TCH = 16


def _tc_pack(codebooks, S, K, D):
    KB = 128

    def body(x_ref, o_ref):
        bf = x_ref[0].astype(jnp.bfloat16)
        u = lax.bitcast_convert_type(bf, jnp.uint16).astype(jnp.uint32)
        ur = u.reshape(KB, D // 2, 2)
        word = ur[:, :, 0] | (ur[:, :, 1] << 16)
        o_ref[0] = lax.bitcast_convert_type(word, jnp.int32)

    return pl.pallas_call(
        body,
        grid=(S, K // KB),
        in_specs=[pl.BlockSpec((1, KB, D), lambda s, k: (s, k, 0))],
        out_specs=pl.BlockSpec((1, KB, D // 2), lambda s, k: (s, k, 0))
        ,
        out_shape=jax.ShapeDtypeStruct((S, K, D // 2), jnp.int32),
    )(codebooks)


def _sc_decode(widx, cbw, S, D, P):
    NCH = P // TCH
    W = D // 2
    mesh = plsc.VectorSubcoreMesh(core_axis_name="c", subcore_axis_name="s")

    @functools.partial(
        pl.kernel,
        out_type=jax.ShapeDtypeStruct((NW * P, W), jnp.int32),
        mesh=mesh,
        compiler_params=pltpu.CompilerParams(needs_layout_passes=False),
        scratch_types=[
            pltpu.VMEM((NCH * S, TCH), jnp.int32),
            pltpu.VMEM((2, S * TCH, W), jnp.int32),
            pltpu.SemaphoreType.DMA,
            pltpu.SemaphoreType.DMA,
            pltpu.SemaphoreType.DMA,
            pltpu.SemaphoreType.DMA,
        ],
    )
    def sc_decode(
        idx_hbm, cb_hbm, out_hbm, idx_v, sbuf, sem0, sem1, osem0, osem1
    ):
        sems = (sem0, sem1)
        osems = (osem0, osem1)
        w = lax.axis_index("s") * NC + lax.axis_index("c")
        pltpu.sync_copy(idx_hbm.at[w], idx_v)

        def fire(c, par):
            for s in range(S):
                pltpu.async_copy(
                    cb_hbm.at[idx_v.at[c * S + s]],
                    sbuf.at[par, pl.ds(s * TCH, TCH)],
                    sems[par],
                )

        def drain(par):
            pltpu.make_async_copy(
                cb_hbm.at[pl.ds(0, S * TCH)], sbuf.at[par], sems[par]
            ).wait()

        def wait_out(par):
            pltpu.make_async_copy(
                sbuf.at[par, pl.ds(0, TCH)],
                out_hbm.at[pl.ds(0, TCH)],
                osems[par],
            ).wait()

        fire(0, 0)

        def outer(cc, carry):
            for par in range(2):
                c = cc * 2 + par
                cn = jnp.minimum(c + 1, NCH - 1)
                if par == 0:
                    @pl.when(cc > 0)
                    def _():
                        wait_out(1 - par)
                else:
                    wait_out(1 - par)
                fire(cn, 1 - par)
                drain(par)

                @plsc.parallel_loop(0, TCH, unroll=2)
                def t_body(t):
                    for k in range(W // 16):
                        sl = pl.ds(k * 16, 16)

                        def ld(s):
                            return plsc.bitcast(
                                sbuf[par, s * TCH + t, sl], jnp.bfloat16
                            )

                        v01 = ld(0) + ld(1)
                        v23 = ld(2) + ld(3)
                        v45 = ld(4) + ld(5)
                        v67 = ld(6) + ld(7)
                        r = (v01 + v23) + (v45 + v67)
                        sbuf[par, t, sl] = plsc.bitcast(r, jnp.int32)

                pltpu.async_copy(
                    sbuf.at[par, pl.ds(0, TCH)],
                    out_hbm.at[pl.ds(w * P + c * TCH, TCH)],
                    osems[par],
                )
            return carry

        lax.fori_loop(0, NCH // 2, outer, 0)
        drain(0)
        wait_out(1)

    return sc_decode(widx, cbw)


def _tc_unpack_transpose(tmp, B, T, D):
    W = D // 2

    TT = 512

    def body(x_ref, o_ref):
        wrd = x_ref[0]
        lo = lax.bitcast_convert_type(
            lax.shift_left(wrd, 16), jnp.float32
        )
        hi = lax.bitcast_convert_type(
            wrd & jnp.int32(-65536), jnp.float32
        )
        comb = jnp.stack([lo, hi], axis=2).reshape(TT, D)
        o_ref[0] = jnp.swapaxes(comb, 0, 1)

    return pl.pallas_call(
        body,
        grid=(B, T // TT),
        in_specs=[pl.BlockSpec((1, TT, W), lambda b, t: (b, t, 0))],
        out_specs=pl.BlockSpec((1, D, TT), lambda b, t: (b, 0, t)),
        out_shape=jax.ShapeDtypeStruct((B, D, T), jnp.float32),
    )(tmp)


def kernel(stage_indices, codebooks):
    S, K, D = codebooks.shape
    _, B, T = stage_indices.shape
    P = B * T // NW
    NCH = P // TCH

    cbw = _tc_pack(codebooks, S, K, D).reshape(S * K, D // 2)
    idx = stage_indices.astype(jnp.int32) + (
        jnp.arange(S, dtype=jnp.int32) * K
    )[:, None, None]
    widx = (
        idx.transpose(1, 2, 0)
        .reshape(NW, NCH, TCH, S)
        .transpose(0, 1, 3, 2)
        .reshape(NW, NCH * S, TCH)
    )

    tmp = _sc_decode(widx, cbw, S, D, P)
    return _tc_unpack_transpose(tmp.reshape(B, T, D // 2), B, T, D)


# speedup vs baseline: 13.1660x; 13.1660x over previous
import functools

import jax
import jax.numpy as jnp
from jax import lax
from jax.experimental import pallas as pl
from jax.experimental.pallas import tpu as pltpu
from jax.experimental.pallas import tpu_sc as plsc

NC = 2
NS = 16
NW = NC * NS
TCH = 16


def _tc_pack(codebooks, S, K, D):
    KB = 256
    W = D // 2

    def body(x_ref, o_ref):
        bf = x_ref[0].astype(jnp.bfloat16)
        u = lax.bitcast_convert_type(bf, jnp.uint16).astype(jnp.uint32)
        word = u[:, :W] | (u[:, W:] << 16)
        o_ref[0] = lax.bitcast_convert_type(word, jnp.int32)

    return pl.pallas_call(
        body,
        grid=(S, K // KB),
        in_specs=[pl.BlockSpec((1, KB, D), lambda s, k: (s, k, 0))],
        out_specs=pl.BlockSpec((1, KB, W), lambda s, k: (s, k, 0)),
        out_shape=jax.ShapeDtypeStruct((S, K, W), jnp.int32),
    )(codebooks)


def _sc_decode(widx, cbw, S, D, P):
    NCH = P // TCH
    W = D // 2
    mesh = plsc.VectorSubcoreMesh(core_axis_name="c", subcore_axis_name="s")

    @functools.partial(
        pl.kernel,
        out_type=jax.ShapeDtypeStruct((NW * P, W), jnp.int32),
        mesh=mesh,
        compiler_params=pltpu.CompilerParams(needs_layout_passes=False),
        scratch_types=[
            pltpu.VMEM((NCH * S, TCH), jnp.int32),
            pltpu.VMEM((2, S * TCH, W), jnp.int32),
            pltpu.SemaphoreType.DMA,
            pltpu.SemaphoreType.DMA,
            pltpu.SemaphoreType.DMA,
            pltpu.SemaphoreType.DMA,
        ],
    )
    def sc_decode(
        idx_hbm, cb_hbm, out_hbm, idx_v, sbuf, sem0, sem1, osem0, osem1
    ):
        sems = (sem0, sem1)
        osems = (osem0, osem1)
        w = lax.axis_index("s") * NC + lax.axis_index("c")
        pltpu.sync_copy(idx_hbm.at[w], idx_v)

        def fire(c, par):
            for s in range(S):
                pltpu.async_copy(
                    cb_hbm.at[idx_v.at[c * S + s]],
                    sbuf.at[par, pl.ds(s * TCH, TCH)],
                    sems[par],
                )

        def drain(par):
            pltpu.make_async_copy(
                cb_hbm.at[pl.ds(0, S * TCH)], sbuf.at[par], sems[par]
            ).wait()

        def wait_out(par):
            pltpu.make_async_copy(
                sbuf.at[par, pl.ds(0, TCH)],
                out_hbm.at[pl.ds(0, TCH)],
                osems[par],
            ).wait()

        fire(0, 0)

        def outer(cc, carry):
            for par in range(2):
                c = cc * 2 + par
                cn = jnp.minimum(c + 1, NCH - 1)
                if par == 0:
                    @pl.when(cc > 0)
                    def _():
                        wait_out(1 - par)
                else:
                    wait_out(1 - par)
                fire(cn, 1 - par)
                drain(par)

                @plsc.parallel_loop(0, TCH, unroll=2)
                def t_body(t):
                    for k in range(W // 16):
                        sl = pl.ds(k * 16, 16)

                        def ld(s):
                            return plsc.bitcast(
                                sbuf[par, s * TCH + t, sl], jnp.bfloat16
                            )

                        v01 = ld(0) + ld(1)
                        v23 = ld(2) + ld(3)
                        v45 = ld(4) + ld(5)
                        v67 = ld(6) + ld(7)
                        r = (v01 + v23) + (v45 + v67)
                        sbuf[par, t, sl] = plsc.bitcast(r, jnp.int32)

                pltpu.async_copy(
                    sbuf.at[par, pl.ds(0, TCH)],
                    out_hbm.at[pl.ds(w * P + c * TCH, TCH)],
                    osems[par],
                )
            return carry

        lax.fori_loop(0, NCH // 2, outer, 0)
        drain(0)
        wait_out(1)

    return sc_decode(widx, cbw)


def _tc_unpack_transpose(tmp, B, T, D):
    W = D // 2

    TT = 512

    def body(x_ref, o_ref):
        wrd = x_ref[0]
        lo = lax.bitcast_convert_type(
            lax.shift_left(wrd, 16), jnp.float32
        )
        hi = lax.bitcast_convert_type(
            wrd & jnp.int32(-65536), jnp.float32
        )
        o_ref[0, :W, :] = jnp.swapaxes(lo, 0, 1)
        o_ref[0, W:, :] = jnp.swapaxes(hi, 0, 1)

    return pl.pallas_call(
        body,
        grid=(B, T // TT),
        in_specs=[pl.BlockSpec((1, TT, W), lambda b, t: (b, t, 0))],
        out_specs=pl.BlockSpec((1, D, TT), lambda b, t: (b, 0, t)),
        out_shape=jax.ShapeDtypeStruct((B, D, T), jnp.float32),
    )(tmp)


def kernel(stage_indices, codebooks):
    S, K, D = codebooks.shape
    _, B, T = stage_indices.shape
    P = B * T // NW
    NCH = P // TCH

    cbw = _tc_pack(codebooks, S, K, D).reshape(S * K, D // 2)
    idx = stage_indices.astype(jnp.int32) + (
        jnp.arange(S, dtype=jnp.int32) * K
    )[:, None, None]
    widx = (
        idx.transpose(1, 2, 0)
        .reshape(NW, NCH, TCH, S)
        .transpose(0, 1, 3, 2)
        .reshape(NW, NCH * S, TCH)
    )

    tmp = _sc_decode(widx, cbw, S, D, P)
    return _tc_unpack_transpose(tmp.reshape(B, T, D // 2), B, T, D)


# speedup vs baseline: 15.2147x; 1.1556x over previous
import functools

import jax
import jax.numpy as jnp
from jax import lax
from jax.experimental import pallas as pl
from jax.experimental.pallas import tpu as pltpu
from jax.experimental.pallas import tpu_sc as plsc

NC = 2
NS = 16
NW = NC * NS
TCH = 16


def _tc_pack(codebooks, S, K, D):
    KB = 256
    W = D // 2

    def body(x_ref, o_ref):
        bf = x_ref[0].astype(jnp.bfloat16)
        u = lax.bitcast_convert_type(bf, jnp.uint16).astype(jnp.uint32)
        word = u[:, :W] | (u[:, W:] << 16)
        o_ref[0] = lax.bitcast_convert_type(word, jnp.int32)

    return pl.pallas_call(
        body,
        grid=(S, K // KB),
        in_specs=[pl.BlockSpec((1, KB, D), lambda s, k: (s, k, 0))],
        out_specs=pl.BlockSpec((1, KB, W), lambda s, k: (s, k, 0)),
        out_shape=jax.ShapeDtypeStruct((S, K, W), jnp.int32),
    )(codebooks)


def _sc_decode(widx, cbw, S, D, P):
    NCH = P // TCH
    W = D // 2
    mesh = plsc.VectorSubcoreMesh(core_axis_name="c", subcore_axis_name="s")

    @functools.partial(
        pl.kernel,
        out_type=jax.ShapeDtypeStruct((NW * P, W), jnp.int32),
        mesh=mesh,
        compiler_params=pltpu.CompilerParams(needs_layout_passes=False),
        scratch_types=[
            pltpu.VMEM((NCH * S, TCH), jnp.int32),
            pltpu.VMEM((2, S * TCH, W), jnp.int32),
            pltpu.SemaphoreType.DMA,
            pltpu.SemaphoreType.DMA,
            pltpu.SemaphoreType.DMA,
            pltpu.SemaphoreType.DMA,
        ],
    )
    def sc_decode(
        idx_hbm, cb_hbm, out_hbm, idx_v, sbuf, sem0, sem1, osem0, osem1
    ):
        sems = (sem0, sem1)
        osems = (osem0, osem1)
        w = lax.axis_index("s") * NC + lax.axis_index("c")
        pltpu.sync_copy(idx_hbm.at[w], idx_v)

        def fire(c, par):
            for s in range(S):
                pltpu.async_copy(
                    cb_hbm.at[idx_v.at[c * S + s]],
                    sbuf.at[par, pl.ds(s * TCH, TCH)],
                    sems[par],
                )

        def drain(par):
            pltpu.make_async_copy(
                cb_hbm.at[pl.ds(0, S * TCH)], sbuf.at[par], sems[par]
            ).wait()

        def wait_out(par):
            pltpu.make_async_copy(
                sbuf.at[par, pl.ds(0, TCH)],
                out_hbm.at[pl.ds(0, TCH)],
                osems[par],
            ).wait()

        fire(0, 0)

        def outer(cc, carry):
            for par in range(2):
                c = cc * 2 + par
                cn = jnp.minimum(c + 1, NCH - 1)
                if par == 0:
                    @pl.when(cc > 0)
                    def _():
                        wait_out(1 - par)
                else:
                    wait_out(1 - par)
                fire(cn, 1 - par)
                drain(par)

                @plsc.parallel_loop(0, TCH, unroll=2)
                def t_body(t):
                    for k in range(W // 16):
                        sl = pl.ds(k * 16, 16)

                        def ld(s):
                            return plsc.bitcast(
                                sbuf[par, s * TCH + t, sl], jnp.bfloat16
                            )

                        v01 = ld(0) + ld(1)
                        v23 = ld(2) + ld(3)
                        v45 = ld(4) + ld(5)
                        v67 = ld(6) + ld(7)
                        r = (v01 + v23) + (v45 + v67)
                        sbuf[par, t, sl] = plsc.bitcast(r, jnp.int32)

                pltpu.async_copy(
                    sbuf.at[par, pl.ds(0, TCH)],
                    out_hbm.at[pl.ds(w * P + c * TCH, TCH)],
                    osems[par],
                )
            return carry

        lax.fori_loop(0, NCH // 2, outer, 0)
        drain(0)
        wait_out(1)

    return sc_decode(widx, cbw)


def _tc_unpack_transpose(tmp, B, T, D):
    W = D // 2

    TT = 2048

    def body(x_ref, o_ref):
        wrd = x_ref[0]
        lo = lax.bitcast_convert_type(
            lax.shift_left(wrd, 16), jnp.float32
        )
        hi = lax.bitcast_convert_type(
            wrd & jnp.int32(-65536), jnp.float32
        )
        o_ref[0, :W, :] = jnp.swapaxes(lo, 0, 1)
        o_ref[0, W:, :] = jnp.swapaxes(hi, 0, 1)

    return pl.pallas_call(
        body,
        grid=(B, T // TT),
        in_specs=[pl.BlockSpec((1, TT, W), lambda b, t: (b, t, 0))],
        out_specs=pl.BlockSpec((1, D, TT), lambda b, t: (b, 0, t)),
        out_shape=jax.ShapeDtypeStruct((B, D, T), jnp.float32),
    )(tmp)


def kernel(stage_indices, codebooks):
    S, K, D = codebooks.shape
    _, B, T = stage_indices.shape
    P = B * T // NW
    NCH = P // TCH

    cbw = _tc_pack(codebooks, S, K, D).reshape(S * K, D // 2)
    idx = stage_indices.astype(jnp.int32) + (
        jnp.arange(S, dtype=jnp.int32) * K
    )[:, None, None]
    widx = (
        idx.transpose(1, 2, 0)
        .reshape(NW, NCH, TCH, S)
        .transpose(0, 1, 3, 2)
        .reshape(NW, NCH * S, TCH)
    )

    tmp = _sc_decode(widx, cbw, S, D, P)
    return _tc_unpack_transpose(tmp.reshape(B, T, D // 2), B, T, D)


# speedup vs baseline: 16.3647x; 1.0756x over previous
import functools

import jax
import jax.numpy as jnp
from jax import lax
from jax.experimental import pallas as pl
from jax.experimental.pallas import tpu as pltpu
from jax.experimental.pallas import tpu_sc as plsc

NC = 2
NS = 16
NW = NC * NS
TCH = 16


def _tc_pack(codebooks, S, K, D):
    KB = 1024
    W = D // 2

    def body(x_ref, o_ref):
        bf = x_ref[0].astype(jnp.bfloat16)
        u = lax.bitcast_convert_type(bf, jnp.uint16).astype(jnp.uint32)
        word = u[:, :W] | (u[:, W:] << 16)
        o_ref[0] = lax.bitcast_convert_type(word, jnp.int32)

    return pl.pallas_call(
        body,
        grid=(S, K // KB),
        in_specs=[pl.BlockSpec((1, KB, D), lambda s, k: (s, k, 0))],
        out_specs=pl.BlockSpec((1, KB, W), lambda s, k: (s, k, 0)),
        out_shape=jax.ShapeDtypeStruct((S, K, W), jnp.int32),
    )(codebooks)


def _sc_decode(widx, cbw, S, D, P):
    NCH = P // TCH
    W = D // 2
    mesh = plsc.VectorSubcoreMesh(core_axis_name="c", subcore_axis_name="s")

    @functools.partial(
        pl.kernel,
        out_type=jax.ShapeDtypeStruct((NW * P, W), jnp.int32),
        mesh=mesh,
        compiler_params=pltpu.CompilerParams(needs_layout_passes=False),
        scratch_types=[
            pltpu.VMEM((NCH * S, TCH), jnp.int32),
            pltpu.VMEM((2, S * TCH, W), jnp.int32),
            pltpu.SemaphoreType.DMA,
            pltpu.SemaphoreType.DMA,
            pltpu.SemaphoreType.DMA,
            pltpu.SemaphoreType.DMA,
        ],
    )
    def sc_decode(
        idx_hbm, cb_hbm, out_hbm, idx_v, sbuf, sem0, sem1, osem0, osem1
    ):
        sems = (sem0, sem1)
        osems = (osem0, osem1)
        w = lax.axis_index("s") * NC + lax.axis_index("c")
        pltpu.sync_copy(idx_hbm.at[w], idx_v)

        def fire(c, par):
            for s in range(S):
                pltpu.async_copy(
                    cb_hbm.at[idx_v.at[c * S + s]],
                    sbuf.at[par, pl.ds(s * TCH, TCH)],
                    sems[par],
                )

        def drain(par):
            pltpu.make_async_copy(
                cb_hbm.at[pl.ds(0, S * TCH)], sbuf.at[par], sems[par]
            ).wait()

        def wait_out(par):
            pltpu.make_async_copy(
                sbuf.at[par, pl.ds(0, TCH)],
                out_hbm.at[pl.ds(0, TCH)],
                osems[par],
            ).wait()

        fire(0, 0)

        def outer(cc, carry):
            for par in range(2):
                c = cc * 2 + par
                cn = jnp.minimum(c + 1, NCH - 1)
                if par == 0:
                    @pl.when(cc > 0)
                    def _():
                        wait_out(1 - par)
                else:
                    wait_out(1 - par)
                fire(cn, 1 - par)
                drain(par)

                @plsc.parallel_loop(0, TCH, unroll=2)
                def t_body(t):
                    for k in range(W // 16):
                        sl = pl.ds(k * 16, 16)

                        def ld(s):
                            return plsc.bitcast(
                                sbuf[par, s * TCH + t, sl], jnp.bfloat16
                            )

                        v01 = ld(0) + ld(1)
                        v23 = ld(2) + ld(3)
                        v45 = ld(4) + ld(5)
                        v67 = ld(6) + ld(7)
                        r = (v01 + v23) + (v45 + v67)
                        sbuf[par, t, sl] = plsc.bitcast(r, jnp.int32)

                pltpu.async_copy(
                    sbuf.at[par, pl.ds(0, TCH)],
                    out_hbm.at[pl.ds(w * P + c * TCH, TCH)],
                    osems[par],
                )
            return carry

        lax.fori_loop(0, NCH // 2, outer, 0)
        drain(0)
        wait_out(1)

    return sc_decode(widx, cbw)


def _tc_unpack_transpose(tmp, B, T, D):
    W = D // 2

    TT = 2048

    def body(x_ref, o_ref):
        wrd = x_ref[0]
        lo = lax.bitcast_convert_type(
            lax.shift_left(wrd, 16), jnp.float32
        )
        hi = lax.bitcast_convert_type(
            wrd & jnp.int32(-65536), jnp.float32
        )
        o_ref[0, :W, :] = jnp.swapaxes(lo, 0, 1)
        o_ref[0, W:, :] = jnp.swapaxes(hi, 0, 1)

    return pl.pallas_call(
        body,
        grid=(B, T // TT),
        in_specs=[pl.BlockSpec((1, TT, W), lambda b, t: (b, t, 0))],
        out_specs=pl.BlockSpec((1, D, TT), lambda b, t: (b, 0, t)),
        out_shape=jax.ShapeDtypeStruct((B, D, T), jnp.float32),
    )(tmp)


def kernel(stage_indices, codebooks):
    S, K, D = codebooks.shape
    _, B, T = stage_indices.shape
    P = B * T // NW
    NCH = P // TCH

    cbw = _tc_pack(codebooks, S, K, D).reshape(S * K, D // 2)
    idx = stage_indices.astype(jnp.int32) + (
        jnp.arange(S, dtype=jnp.int32) * K
    )[:, None, None]
    widx = (
        idx.transpose(1, 2, 0)
        .reshape(NW, NCH, TCH, S)
        .transpose(0, 1, 3, 2)
        .reshape(NW, NCH * S, TCH)
    )

    tmp = _sc_decode(widx, cbw, S, D, P)
    return _tc_unpack_transpose(tmp.reshape(B, T, D // 2), B, T, D)
